# Initial kernel scaffold; baseline (speedup 1.0000x reference)
import jax
import jax.numpy as jnp
from jax.experimental import pallas as pl


def kernel(src_p3_camera, src_p3_lidar, src_p4_camera, src_p4_lidar, src_p5_camera, src_p5_lidar, w1_p3_camera, w2_p3_camera, w1_p3_lidar, w2_p3_lidar, w1_p4_camera, w2_p4_camera, w1_p4_lidar, w2_p4_lidar):
    raise NotImplementedError("write your pallas kernel here")


# speedup vs baseline: 2.0370x; 2.0370x over previous
import functools

import jax
import jax.numpy as jnp
from jax.experimental import pallas as pl
from jax.experimental.pallas import tpu as pltpu


def _adapter_kernel(x_ref, w1_ref, w2_ref, o_ref, *, res_scale):
    x = x_ref[0]
    h = jnp.maximum(jnp.dot(w1_ref[...], x, preferred_element_type=jnp.float32),
                    0.0)
    y = jnp.maximum(jnp.dot(w2_ref[...], h, preferred_element_type=jnp.float32),
                    0.0)
    o_ref[0] = (y + res_scale * x).astype(o_ref.dtype)


def _adapt_one(feat, w1, w2, ratio, *, tile=2048):
    bs, c, h, w = feat.shape
    hw = h * w
    t = min(tile, hw)
    n_tiles = hw // t
    assert n_tiles * t == hw, (hw, t)
    x = feat.reshape(bs, c, hw)
    w2r = w2.astype(jnp.float32) * jnp.float32(ratio)

    out = pl.pallas_call(
        functools.partial(_adapter_kernel, res_scale=1.0 - float(ratio)),
        out_shape=jax.ShapeDtypeStruct((bs, c, hw), feat.dtype),
        grid=(bs, n_tiles),
        in_specs=[
            pl.BlockSpec((1, c, t), lambda b, j: (b, 0, j)),
            pl.BlockSpec(w1.shape, lambda b, j: (0, 0)),
            pl.BlockSpec(w2.shape, lambda b, j: (0, 0)),
        ],
        out_specs=pl.BlockSpec((1, c, t), lambda b, j: (b, 0, j)),
        compiler_params=pltpu.CompilerParams(
            dimension_semantics=("parallel", "parallel"),
        ),
    )(x, w1.astype(jnp.float32), w2r)
    return out.reshape(bs, c, h, w)


def kernel(src_p3_camera, src_p3_lidar, src_p4_camera, src_p4_lidar,
           src_p5_camera, src_p5_lidar,
           w1_p3_camera, w2_p3_camera, w1_p3_lidar, w2_p3_lidar,
           w1_p4_camera, w2_p4_camera, w1_p4_lidar, w2_p4_lidar):
    r_cam, r_lid = 0.2, 0.6
    return {
        "p3": {
            "camera": _adapt_one(src_p3_camera, w1_p3_camera, w2_p3_camera,
                                 r_cam),
            "lidar": _adapt_one(src_p3_lidar, w1_p3_lidar, w2_p3_lidar, r_lid),
        },
        "p4": {
            "camera": _adapt_one(src_p4_camera, w1_p4_camera, w2_p4_camera,
                                 r_cam),
            "lidar": _adapt_one(src_p4_lidar, w1_p4_lidar, w2_p4_lidar, r_lid),
        },
        "p5": {"camera": src_p5_camera, "lidar": src_p5_lidar},
    }
